# baseline (speedup 1.0000x reference)
import functools

import jax
import jax.numpy as jnp
from jax import lax
from jax.experimental import pallas as pl
from jax.experimental.pallas import tpu as pltpu
from jax.experimental.pallas import tpu_sc as plsc

N_COLS = 22
VOCAB = 100000
EMB = 32
BATCH = 16384
TOTAL = BATCH * N_COLS
NUM_WORKERS = 32
PER_WORKER = TOTAL // NUM_WORKERS
CHUNK = 1024
NUM_CHUNKS = PER_WORKER // CHUNK

assert TOTAL % NUM_WORKERS == 0
assert PER_WORKER % CHUNK == 0

_mesh = plsc.VectorSubcoreMesh(core_axis_name="c", subcore_axis_name="s")


@functools.partial(
    pl.kernel,
    mesh=_mesh,
    compiler_params=pltpu.CompilerParams(use_tc_tiling_on_sc=False),
    out_type=jax.ShapeDtypeStruct((TOTAL, 2, EMB), jnp.float32),
    scratch_types=[
        pltpu.VMEM((CHUNK,), jnp.int32),
        pltpu.VMEM((CHUNK, EMB), jnp.float32),
        pltpu.VMEM((CHUNK, EMB), jnp.float32),
        pltpu.SemaphoreType.DMA,
        pltpu.SemaphoreType.DMA,
    ],
)
def _gather_kernel(idx_hbm, mean_hbm, std_hbm, out_hbm,
                   idx_v, mean_v, std_v, sem_m, sem_s):
    wid = lax.axis_index("s") * 2 + lax.axis_index("c")
    base = wid * PER_WORKER

    def body(i, carry):
        off = base + i * CHUNK
        pltpu.sync_copy(idx_hbm.at[pl.ds(off, CHUNK)], idx_v)
        cm = pltpu.async_copy(mean_hbm.at[idx_v], mean_v, sem_m)
        cs = pltpu.async_copy(std_hbm.at[idx_v], std_v, sem_s)
        cm.wait()
        cs.wait()
        pltpu.sync_copy(mean_v, out_hbm.at[pl.ds(off, CHUNK), 0])
        pltpu.sync_copy(std_v, out_hbm.at[pl.ds(off, CHUNK), 1])
        return carry

    lax.fori_loop(0, NUM_CHUNKS, body, 0)


def kernel(features, emb_mean, emb_std):
    flat_idx = (features.astype(jnp.int32)
                + (jnp.arange(N_COLS, dtype=jnp.int32) * VOCAB)[None, :])
    flat_idx = flat_idx.reshape(TOTAL)
    mean2d = emb_mean.reshape(N_COLS * VOCAB, EMB)
    std2d = emb_std.reshape(N_COLS * VOCAB, EMB)
    out = _gather_kernel(flat_idx, mean2d, std2d)
    return out.reshape(BATCH, N_COLS, 2 * EMB)
